# baseline (speedup 1.0000x reference)
import jax
import jax.numpy as jnp
from jax import lax
from jax.experimental import pallas as pl
from jax.experimental.pallas import tpu as pltpu
from jax.experimental.pallas import tpu_sc as plsc

B, H, D = 4096, 200, 64
NV = 1000
ROWS = B * H
NC, NS = 2, 16
NW = NC * NS
RPW = ROWS // NW
C = 128
IROWS = ROWS // C
IRPW = IROWS // NW
NBUF = 2
T = IRPW // NBUF


def _fill_verbs(vidx_v, verbs_v, vbuf, g):
    lanes = lax.iota(jnp.int32, 16)
    for r16 in range(C // 16):
        idx16 = vidx_v[pl.ds(g * C + r16 * 16, 16)]
        rows16 = lanes + (r16 * 16)

        def col_body(w, _, idx16=idx16, rows16=rows16):
            wsplat = jnp.full((16,), 0, jnp.int32) + w
            vals = plsc.load_gather(verbs_v, [idx16, wsplat])
            plsc.store_scatter(vbuf, [rows16, wsplat], vals)
            return 0

        lax.fori_loop(0, D, col_body, 0, unroll=8)


def _body(vidx_hbm, nidx_hbm, verbs_hbm, nouns_hbm, out_hbm,
          verbs_v, vidx_v, nidx_v, vb0, vb1, nb0, nb1,
          sem_i, sem_g, sem_wv, sem_wn):
    vbufs = [vb0, vb1]
    nbufs = [nb0, nb1]
    wid = lax.axis_index("s") * NC + lax.axis_index("c")
    row0 = wid * RPW
    irow0 = wid * IRPW

    pltpu.sync_copy(verbs_hbm, verbs_v)
    pltpu.sync_copy(vidx_hbm.at[pl.ds(wid * RPW, RPW)], vidx_v)
    pltpu.sync_copy(nidx_hbm.at[pl.ds(irow0, NBUF)], nidx_v.at[0])

    def wait_write_v(j):
        pltpu.make_async_copy(
            vbufs[j], out_hbm.at[pl.ds(0, C), pl.ds(0, D)], sem_wv.at[j]).wait()

    def wait_write_n(j):
        pltpu.make_async_copy(
            nbufs[j], out_hbm.at[pl.ds(0, C), pl.ds(D, D)], sem_wn.at[j]).wait()

    def block(t, carry):
        p = lax.rem(t, 2)
        pn = 1 - p

        @pl.when(t > 0)
        def _():
            pltpu.make_async_copy(
                nidx_hbm.at[pl.ds(irow0, NBUF)], nidx_v.at[p], sem_i.at[p]).wait()
            for j in range(NBUF):
                wait_write_n(j)

        gathers = []
        for j in range(NBUF):
            gathers.append(pltpu.async_copy(
                nouns_hbm.at[nidx_v.at[p, j]], nbufs[j], sem_g.at[j]))

        @pl.when(t + 1 < T)
        def _():
            pltpu.async_copy(
                nidx_hbm.at[pl.ds(irow0 + (t + 1) * NBUF, NBUF)],
                nidx_v.at[pn], sem_i.at[pn])

        for j in range(NBUF):
            @pl.when(t > 0)
            def _(j=j):
                wait_write_v(j)
            _fill_verbs(vidx_v, verbs_v, vbufs[j], t * NBUF + j)

        for j in range(NBUF):
            g = t * NBUF + j
            r0 = row0 + g * C
            gathers[j].wait()
            pltpu.async_copy(
                vbufs[j], out_hbm.at[pl.ds(r0, C), pl.ds(0, D)], sem_wv.at[j])
            pltpu.async_copy(
                nbufs[j], out_hbm.at[pl.ds(r0, C), pl.ds(D, D)], sem_wn.at[j])
        return carry

    lax.fori_loop(0, T, block, 0)
    for j in range(NBUF):
        wait_write_v(j)
        wait_write_n(j)


@jax.jit
def _run(vidx, nidx, verbs_table, nouns_table):
    fn = pl.kernel(
        _body,
        out_type=jax.ShapeDtypeStruct((ROWS, 2 * D), jnp.float32),
        mesh=plsc.VectorSubcoreMesh(core_axis_name="c", subcore_axis_name="s"),
        compiler_params=pltpu.CompilerParams(
            use_tc_tiling_on_sc=False, needs_layout_passes=False),
        scratch_types=(
            [
                pltpu.VMEM((NV, D), jnp.float32),
                pltpu.VMEM((RPW,), jnp.int32),
                pltpu.VMEM((2, NBUF, C), jnp.int32),
            ]
            + [pltpu.VMEM((C, D), jnp.float32)] * (2 * NBUF)
            + [pltpu.SemaphoreType.DMA((2,))]
            + [pltpu.SemaphoreType.DMA((NBUF,))] * 3
        ),
    )
    return fn(vidx, nidx, verbs_table, nouns_table)


def kernel(observed_labels, verbs_table, nouns_table):
    vidx = observed_labels[:, :, 0].reshape(ROWS)
    nidx = observed_labels[:, :, 1].reshape(IROWS, C)
    out = _run(vidx, nidx, verbs_table, nouns_table)
    return out.reshape(B, H, 2 * D)


# speedup vs baseline: 1.3741x; 1.3741x over previous
import jax
import jax.numpy as jnp
from jax import lax
from jax.experimental import pallas as pl
from jax.experimental.pallas import tpu as pltpu
from jax.experimental.pallas import tpu_sc as plsc

B, H, D = 4096, 200, 64
ROWS = B * H
NC, NS = 2, 16
NW = NC * NS
RPW = ROWS // NW
C = 128
CHUNKS = RPW // C
NBUF = 4
T = CHUNKS // NBUF


def _body(labels_hbm, verbs_hbm, nouns_hbm, out_hbm,
          labels_v, vi0, vi1, vi2, vi3, ni0, ni1, ni2, ni3,
          vb0, vb1, vb2, vb3, nb0, nb1, nb2, nb3,
          sem_gv, sem_gn, sem_wv, sem_wn):
    vidx = [vi0, vi1, vi2, vi3]
    nidx = [ni0, ni1, ni2, ni3]
    vbufs = [vb0, vb1, vb2, vb3]
    nbufs = [nb0, nb1, nb2, nb3]
    wid = lax.axis_index("s") * NC + lax.axis_index("c")
    row0 = wid * RPW

    pltpu.sync_copy(labels_hbm.at[pl.ds(2 * row0, 2 * RPW)], labels_v)

    lanes2 = lax.iota(jnp.int32, 16) * 2

    def build_idx(g, j):
        for r16 in range(C // 16):
            base = jnp.full((16,), 0, jnp.int32) + (g * (2 * C) + r16 * 32)
            vidx[j][pl.ds(r16 * 16, 16)] = plsc.load_gather(
                labels_v, [base + lanes2])
            nidx[j][pl.ds(r16 * 16, 16)] = plsc.load_gather(
                labels_v, [base + lanes2 + 1])

    def wait_write_v(j):
        pltpu.make_async_copy(
            vbufs[j], out_hbm.at[pl.ds(0, C), pl.ds(0, D)], sem_wv.at[j]).wait()

    def wait_write_n(j):
        pltpu.make_async_copy(
            nbufs[j], out_hbm.at[pl.ds(0, C), pl.ds(D, D)], sem_wn.at[j]).wait()

    def block(t, carry):
        @pl.when(t > 0)
        def _():
            for j in range(NBUF):
                wait_write_v(j)
                wait_write_n(j)
        cps = []
        for j in range(NBUF):
            g = t * NBUF + j
            build_idx(g, j)
            cps.append((
                pltpu.async_copy(verbs_hbm.at[vidx[j]], vbufs[j], sem_gv.at[j]),
                pltpu.async_copy(nouns_hbm.at[nidx[j]], nbufs[j], sem_gn.at[j]),
            ))
        for j in range(NBUF):
            g = t * NBUF + j
            r0 = row0 + g * C
            cps[j][0].wait()
            cps[j][1].wait()
            pltpu.async_copy(vbufs[j], out_hbm.at[pl.ds(r0, C), pl.ds(0, D)], sem_wv.at[j])
            pltpu.async_copy(nbufs[j], out_hbm.at[pl.ds(r0, C), pl.ds(D, D)], sem_wn.at[j])
        return carry

    lax.fori_loop(0, T, block, 0)
    for j in range(NBUF):
        wait_write_v(j)
        wait_write_n(j)


@jax.jit
def _run(labels, verbs_table, nouns_table):
    fn = pl.kernel(
        _body,
        out_type=jax.ShapeDtypeStruct((ROWS, 2 * D), jnp.float32),
        mesh=plsc.VectorSubcoreMesh(core_axis_name="c", subcore_axis_name="s"),
        compiler_params=pltpu.CompilerParams(
            use_tc_tiling_on_sc=False, needs_layout_passes=False),
        scratch_types=(
            [pltpu.VMEM((2 * RPW,), jnp.int32)]
            + [pltpu.VMEM((C,), jnp.int32)] * (2 * NBUF)
            + [pltpu.VMEM((C, D), jnp.float32)] * (2 * NBUF)
            + [pltpu.SemaphoreType.DMA((NBUF,))] * 4
        ),
    )
    return fn(labels, verbs_table, nouns_table)


def kernel(observed_labels, verbs_table, nouns_table):
    labels = observed_labels.reshape(2 * ROWS)
    out = _run(labels, verbs_table, nouns_table)
    return out.reshape(B, H, 2 * D)


# speedup vs baseline: 3.9856x; 2.9006x over previous
import jax
import jax.numpy as jnp
from jax import lax
from jax.experimental import pallas as pl
from jax.experimental.pallas import tpu as pltpu
from jax.experimental.pallas import tpu_sc as plsc

B, H, D = 4096, 200, 64
ROWS = B * H
NC, NS = 2, 16
NW = NC * NS
RPW = ROWS // NW
C = 128
IROWS = ROWS // C
IRPW = IROWS // NW
NBUF = 4
T = IRPW // NBUF


def _body(idx_hbm, verbs_hbm, nouns_hbm, out_hbm,
          vidx_v, nidx_v,
          vb0, vb1, vb2, vb3, nb0, nb1, nb2, nb3,
          sem_gv, sem_gn, sem_wv, sem_wn):
    vbufs = [vb0, vb1, vb2, vb3]
    nbufs = [nb0, nb1, nb2, nb3]
    wid = lax.axis_index("s") * NC + lax.axis_index("c")
    row0 = wid * RPW

    pltpu.sync_copy(idx_hbm.at[0, pl.ds(wid * IRPW, IRPW)], vidx_v)
    pltpu.sync_copy(idx_hbm.at[1, pl.ds(wid * IRPW, IRPW)], nidx_v)

    def wait_write_v(j):
        pltpu.make_async_copy(
            vbufs[j], out_hbm.at[pl.ds(0, C), pl.ds(0, D)], sem_wv.at[j]).wait()

    def wait_write_n(j):
        pltpu.make_async_copy(
            nbufs[j], out_hbm.at[pl.ds(0, C), pl.ds(D, D)], sem_wn.at[j]).wait()

    def block(t, carry):
        @pl.when(t > 0)
        def _():
            for j in range(NBUF):
                wait_write_v(j)
                wait_write_n(j)
        cps = []
        for j in range(NBUF):
            g = t * NBUF + j
            cps.append((
                pltpu.async_copy(verbs_hbm.at[vidx_v.at[g]], vbufs[j], sem_gv.at[j]),
                pltpu.async_copy(nouns_hbm.at[nidx_v.at[g]], nbufs[j], sem_gn.at[j]),
            ))
        for j in range(NBUF):
            g = t * NBUF + j
            r0 = row0 + g * C
            cps[j][0].wait()
            cps[j][1].wait()
            pltpu.async_copy(vbufs[j], out_hbm.at[pl.ds(r0, C), pl.ds(0, D)], sem_wv.at[j])
            pltpu.async_copy(nbufs[j], out_hbm.at[pl.ds(r0, C), pl.ds(D, D)], sem_wn.at[j])
        return carry

    lax.fori_loop(0, T, block, 0)
    for j in range(NBUF):
        wait_write_v(j)
        wait_write_n(j)


@jax.jit
def _run(idx, verbs_table, nouns_table):
    fn = pl.kernel(
        _body,
        out_type=jax.ShapeDtypeStruct((ROWS, 2 * D), jnp.float32),
        mesh=plsc.VectorSubcoreMesh(core_axis_name="c", subcore_axis_name="s"),
        compiler_params=pltpu.CompilerParams(
            use_tc_tiling_on_sc=False, needs_layout_passes=False),
        scratch_types=(
            [pltpu.VMEM((IRPW, C), jnp.int32)] * 2
            + [pltpu.VMEM((C, D), jnp.float32)] * (2 * NBUF)
            + [pltpu.SemaphoreType.DMA((NBUF,))] * 4
        ),
    )
    return fn(idx, verbs_table, nouns_table)


def kernel(observed_labels, verbs_table, nouns_table):
    idx = observed_labels.reshape(ROWS, 2).T.reshape(2, IROWS, C)
    out = _run(idx, verbs_table, nouns_table)
    return out.reshape(B, H, 2 * D)
